# baseline (device time: 13133 ns/iter reference)
import jax
import jax.numpy as jnp
from jax import lax
from jax.experimental import pallas as pl
from jax.experimental.pallas import tpu as pltpu

N_DEV = 4
N_EXPERTS = 16
CAPACITY = 25
E_LOCAL = N_EXPERTS // N_DEV
SLOT_PER_E = 32
N_SLOTS = E_LOCAL * SLOT_PER_E
G_SLOTS = N_DEV * N_SLOTS


def kernel(x, router_W, route_idx, expert_W):
    n, d = x.shape
    h = expert_W.shape[-1]

    def body(x_ref, idx_ref, w_ref, out_ref, comm_ref, send_sems, recv_sems):
        my_pos = lax.axis_index("i")

        barrier_sem = pltpu.get_barrier_semaphore()
        for k in range(1, N_DEV):
            pl.semaphore_signal(
                barrier_sem, inc=1,
                device_id=(lax.rem(my_pos + k, N_DEV),),
                device_id_type=pl.DeviceIdType.MESH,
            )

        e = idx_ref[:, :]
        expert_ids = lax.broadcasted_iota(jnp.int32, (n, N_EXPERTS), 1)
        one_hot = (e == expert_ids).astype(jnp.bfloat16)
        row = lax.broadcasted_iota(jnp.int32, (n, n), 0)
        col = lax.broadcasted_iota(jnp.int32, (n, n), 1)
        strict_lower = (row > col).astype(jnp.bfloat16)
        cum = jnp.dot(strict_lower, one_hot,
                      preferred_element_type=jnp.float32)
        rank_i = jnp.sum(one_hot.astype(jnp.float32) * cum,
                         axis=1, keepdims=True).astype(jnp.int32)
        keep = rank_i < CAPACITY
        owner = lax.div(e, E_LOCAL)
        slot = lax.rem(e, E_LOCAL) * SLOT_PER_E + rank_i

        slot_ids = lax.broadcasted_iota(jnp.int32, (n, N_SLOTS), 1)
        valid_me = jnp.logical_and(keep, owner == my_pos)
        q_me = jnp.logical_and(slot_ids == slot, valid_me).astype(jnp.bfloat16)
        xg = lax.dot_general(
            q_me, x_ref[:, :].astype(jnp.bfloat16),
            (((0,), (0,)), ((), ())),
            preferred_element_type=jnp.float32,
        ).astype(jnp.bfloat16)

        HALF = N_SLOTS // 2
        col_blk = lax.broadcasted_iota(jnp.int32, (N_SLOTS, 2 * d), 1) // d
        row_par = lax.broadcasted_iota(
            jnp.int32, (N_SLOTS, 2 * d), 0) // SLOT_PER_E % 2
        blk_mask = (col_blk == row_par).astype(jnp.bfloat16)
        xg2 = jnp.concatenate([xg, xg], axis=1) * blk_mask
        w_flat = w_ref[:, :, :].astype(jnp.bfloat16).reshape(E_LOCAL * d, h)

        def send_half(hf):
            rdmas = []
            for k in range(1, N_DEV):
                dst = lax.rem(my_pos + k, N_DEV)
                rdma = pltpu.make_async_remote_copy(
                    src_ref=comm_ref.at[my_pos, pl.ds(hf * HALF, HALF)],
                    dst_ref=comm_ref.at[my_pos, pl.ds(hf * HALF, HALF)],
                    send_sem=send_sems.at[hf * (N_DEV - 1) + k - 1],
                    recv_sem=recv_sems.at[my_pos, hf],
                    device_id=(dst,),
                    device_id_type=pl.DeviceIdType.MESH,
                )
                rdma.start()
                rdmas.append(rdma)
            return rdmas

        yg_a = jnp.dot(xg2[:HALF], w_flat[: 2 * d],
                       preferred_element_type=jnp.float32)
        comm_ref[my_pos, pl.ds(0, HALF), :] = yg_a.astype(jnp.bfloat16)
        pl.semaphore_wait(barrier_sem, N_DEV - 1)
        sends = send_half(0)

        yg_b = jnp.dot(xg2[HALF:], w_flat[2 * d:],
                       preferred_element_type=jnp.float32)
        comm_ref[my_pos, pl.ds(HALF, HALF), :] = yg_b.astype(jnp.bfloat16)
        sends += send_half(1)

        out_ref[:, :] = jnp.dot(q_me, comm_ref[my_pos],
                                preferred_element_type=jnp.float32)

        def q_from(c):
            valid = jnp.logical_and(keep, owner == c)
            return jnp.logical_and(slot_ids == slot, valid).astype(jnp.bfloat16)

        order = [1, 3, 2]
        q_srcs = {k: q_from(lax.rem(my_pos + k, N_DEV)) for k in order}

        for hf in range(2):
            for k in order:
                src = lax.rem(my_pos + k, N_DEV)
                recv = pltpu.make_async_remote_copy(
                    src_ref=comm_ref.at[src, pl.ds(hf * HALF, HALF)],
                    dst_ref=comm_ref.at[src, pl.ds(hf * HALF, HALF)],
                    send_sem=send_sems.at[hf * (N_DEV - 1) + k - 1],
                    recv_sem=recv_sems.at[src, hf],
                    device_id=(src,),
                    device_id_type=pl.DeviceIdType.MESH,
                )
                recv.wait_recv()
                out_ref[:, :] = out_ref[:, :] + jnp.dot(
                    q_srcs[k][:, hf * HALF:(hf + 1) * HALF],
                    comm_ref[src, pl.ds(hf * HALF, HALF)],
                    preferred_element_type=jnp.float32,
                )

        for rdma in sends:
            rdma.wait_send()

    return pl.pallas_call(
        body,
        out_shape=jax.ShapeDtypeStruct((n, h), jnp.float32),
        in_specs=[
            pl.BlockSpec(memory_space=pltpu.VMEM),
            pl.BlockSpec(memory_space=pltpu.VMEM),
            pl.BlockSpec(memory_space=pltpu.VMEM),
        ],
        out_specs=pl.BlockSpec(memory_space=pltpu.VMEM),
        scratch_shapes=[
            pltpu.VMEM((N_DEV, N_SLOTS, h), jnp.bfloat16),
            pltpu.SemaphoreType.DMA((2 * (N_DEV - 1),)),
            pltpu.SemaphoreType.DMA((N_DEV, 2)),
        ],
        compiler_params=pltpu.CompilerParams(collective_id=0),
    )(x, route_idx, expert_W)


# device time: 12692 ns/iter; 1.0347x vs baseline; 1.0347x over previous
import jax
import jax.numpy as jnp
from jax import lax
from jax.experimental import pallas as pl
from jax.experimental.pallas import tpu as pltpu

N_DEV = 4
N_EXPERTS = 16
CAPACITY = 25
E_LOCAL = N_EXPERTS // N_DEV
SLOT_PER_E = 28
N_SLOTS = E_LOCAL * SLOT_PER_E
G_SLOTS = N_DEV * N_SLOTS


def kernel(x, router_W, route_idx, expert_W):
    n, d = x.shape
    h = expert_W.shape[-1]

    def body(x_ref, idx_ref, w_ref, out_ref, comm_ref, send_sems, recv_sems):
        my_pos = lax.axis_index("i")

        barrier_sem = pltpu.get_barrier_semaphore()
        for k in range(1, N_DEV):
            pl.semaphore_signal(
                barrier_sem, inc=1,
                device_id=(lax.rem(my_pos + k, N_DEV),),
                device_id_type=pl.DeviceIdType.MESH,
            )

        e = idx_ref[:, :]
        expert_ids = lax.broadcasted_iota(jnp.int32, (n, N_EXPERTS), 1)
        one_hot = (e == expert_ids).astype(jnp.bfloat16)
        row = lax.broadcasted_iota(jnp.int32, (n, n), 0)
        col = lax.broadcasted_iota(jnp.int32, (n, n), 1)
        strict_lower = (row > col).astype(jnp.bfloat16)
        cum = jnp.dot(strict_lower, one_hot,
                      preferred_element_type=jnp.float32)
        rank_i = jnp.sum(one_hot.astype(jnp.float32) * cum,
                         axis=1, keepdims=True).astype(jnp.int32)
        keep = rank_i < CAPACITY
        owner = lax.div(e, E_LOCAL)
        slot = lax.rem(e, E_LOCAL) * SLOT_PER_E + rank_i

        slot_ids = lax.broadcasted_iota(jnp.int32, (n, N_SLOTS), 1)
        valid_me = jnp.logical_and(keep, owner == my_pos)
        q_me = jnp.logical_and(slot_ids == slot, valid_me).astype(jnp.bfloat16)
        xg = lax.dot_general(
            q_me, x_ref[:, :].astype(jnp.bfloat16),
            (((0,), (0,)), ((), ())),
            preferred_element_type=jnp.float32,
        ).astype(jnp.bfloat16)

        col_blk = lax.broadcasted_iota(jnp.int32, (N_SLOTS, E_LOCAL * d), 1) // d
        row_blk = lax.broadcasted_iota(
            jnp.int32, (N_SLOTS, E_LOCAL * d), 0) // SLOT_PER_E
        blk_mask = (col_blk == row_blk).astype(jnp.bfloat16)
        xg_wide = jnp.concatenate([xg] * E_LOCAL, axis=1) * blk_mask
        w_flat = w_ref[:, :, :].astype(jnp.bfloat16).reshape(E_LOCAL * d, h)
        yg = jnp.dot(xg_wide, w_flat,
                     preferred_element_type=jnp.float32)
        comm_ref[my_pos, :, :] = yg.astype(jnp.bfloat16)

        pl.semaphore_wait(barrier_sem, N_DEV - 1)
        sends = []
        for k in range(1, N_DEV):
            dst = lax.rem(my_pos + k, N_DEV)
            rdma = pltpu.make_async_remote_copy(
                src_ref=comm_ref.at[my_pos],
                dst_ref=comm_ref.at[my_pos],
                send_sem=send_sems.at[k - 1],
                recv_sem=recv_sems.at[my_pos],
                device_id=(dst,),
                device_id_type=pl.DeviceIdType.MESH,
            )
            rdma.start()
            sends.append(rdma)

        out_ref[:, :] = jnp.dot(q_me, comm_ref[my_pos],
                                preferred_element_type=jnp.float32)

        def q_from(c):
            valid = jnp.logical_and(keep, owner == c)
            return jnp.logical_and(slot_ids == slot, valid).astype(jnp.bfloat16)

        order = [1, 3, 2]
        q_srcs = {k: q_from(lax.rem(my_pos + k, N_DEV)) for k in order}

        for k in order:
            src = lax.rem(my_pos + k, N_DEV)
            recv = pltpu.make_async_remote_copy(
                src_ref=comm_ref.at[src],
                dst_ref=comm_ref.at[src],
                send_sem=send_sems.at[k - 1],
                recv_sem=recv_sems.at[src],
                device_id=(src,),
                device_id_type=pl.DeviceIdType.MESH,
            )
            recv.wait_recv()
            out_ref[:, :] = out_ref[:, :] + jnp.dot(
                q_srcs[k], comm_ref[src], preferred_element_type=jnp.float32
            )

        for rdma in sends:
            rdma.wait_send()

    return pl.pallas_call(
        body,
        out_shape=jax.ShapeDtypeStruct((n, h), jnp.float32),
        in_specs=[
            pl.BlockSpec(memory_space=pltpu.VMEM),
            pl.BlockSpec(memory_space=pltpu.VMEM),
            pl.BlockSpec(memory_space=pltpu.VMEM),
        ],
        out_specs=pl.BlockSpec(memory_space=pltpu.VMEM),
        scratch_shapes=[
            pltpu.VMEM((N_DEV, N_SLOTS, h), jnp.bfloat16),
            pltpu.SemaphoreType.DMA((N_DEV - 1,)),
            pltpu.SemaphoreType.DMA((N_DEV,)),
        ],
        compiler_params=pltpu.CompilerParams(collective_id=0),
    )(x, route_idx, expert_W)
